# baseline (speedup 1.0000x reference)
import functools

import jax
import jax.numpy as jnp
from jax import lax
from jax.experimental import pallas as pl
from jax.experimental.pallas import tpu as pltpu
from jax.experimental.pallas import tpu_sc as plsc

B = 1024
T = 200
D = 128
V = 1000

NC = 2
NS = 16
NW = NC * NS
BPW = B // NW

DC = 64
NCHUNK = D // DC
LANES = 16
NG = (T + LANES - 1) // LANES
NFULL = NG - 1
TAIL = T - NFULL * LANES
UNROLL = 8


def _make_sc_kernel():
    mesh = plsc.VectorSubcoreMesh(core_axis_name="c", subcore_axis_name="s")

    @functools.partial(
        pl.kernel,
        mesh=mesh,
        out_type=jax.ShapeDtypeStruct((B * D * T,), jnp.float32),
        compiler_params=pltpu.CompilerParams(needs_layout_passes=False),
        scratch_types=[
            pltpu.VMEM((DC * V,), jnp.float32),
            pltpu.VMEM((BPW * T + LANES,), jnp.int32),
            pltpu.VMEM((DC * T,), jnp.float32),
            pltpu.VMEM((DC * T,), jnp.float32),
            pltpu.SemaphoreType.DMA,
            pltpu.SemaphoreType.DMA,
        ],
    )
    def k(idx_hbm, tabt_hbm, out_hbm, tab_v, idx_v, tile0, tile1, sem0, sem1):
        wid = lax.axis_index("s") * NC + lax.axis_index("c")
        lane = lax.iota(jnp.int32, 16)
        vstep = jnp.full((16,), V, jnp.int32)
        tstep = jnp.full((16,), T, jnp.int32)
        tail_mask = lane < TAIL

        idx_v[pl.ds(BPW * T, LANES)] = jnp.zeros((16,), jnp.int32)
        pltpu.sync_copy(
            idx_hbm.at[pl.ds(wid * BPW * T, BPW * T)],
            idx_v.at[pl.ds(0, BPW * T)],
        )

        def drain(tile, sem):
            pltpu.make_async_copy(tile, out_hbm.at[pl.ds(0, DC * T)], sem).wait()

        def gather_tile(i, tile):
            def body_g(g, _):
                idxv = idx_v[pl.ds(i * T + g * LANES, LANES)]

                def body_d(dd, _):
                    vals = plsc.load_gather(tab_v.at[pl.ds(dd * V, V)], [idxv])
                    tile[pl.ds(dd * T + g * LANES, LANES)] = vals
                    return 0

                lax.fori_loop(0, DC, body_d, 0, unroll=UNROLL)
                return 0

            lax.fori_loop(0, NFULL, body_g, 0)

            idxv = idx_v[pl.ds(i * T + NFULL * LANES, LANES)]
            tvec = NFULL * LANES + lane

            def body_dt(dd, _):
                dst = jnp.full((16,), dd * T, jnp.int32) + tvec
                vals = plsc.load_gather(tab_v.at[pl.ds(dd * V, V)], [idxv],
                                        mask=tail_mask)
                plsc.store_scatter(tile, [dst], vals, mask=tail_mask)
                return 0

            lax.fori_loop(0, DC, body_dt, 0, unroll=UNROLL)

        for chunk in range(NCHUNK):
            pltpu.sync_copy(tabt_hbm.at[pl.ds(chunk * DC * V, DC * V)], tab_v)

            def body_v(vv, _, chunk=chunk):
                for p, tile, sem in ((0, tile0, sem0), (1, tile1, sem1)):
                    if chunk == 0:
                        @pl.when(vv > 0)
                        def _():
                            drain(tile, sem)
                    else:
                        drain(tile, sem)
                    i = 2 * vv + p
                    gather_tile(i, tile)
                    b = wid * BPW + i
                    pltpu.async_copy(
                        tile,
                        out_hbm.at[pl.ds((b * D + chunk * DC) * T, DC * T)],
                        sem,
                    )
                return 0

            lax.fori_loop(0, BPW // 2, body_v, 0)

        drain(tile0, sem0)
        drain(tile1, sem1)

    return k


_sc_kernel = _make_sc_kernel()


def _transpose_body(x_ref, o_ref):
    o_ref[...] = jnp.transpose(x_ref[...], (1, 0))


_tc_transpose = pl.pallas_call(
    _transpose_body,
    out_shape=jax.ShapeDtypeStruct((D, V), jnp.float32),
)


def kernel(inputs, embedding_table):
    tabt = _tc_transpose(embedding_table).reshape(D * V)
    out_flat = _sc_kernel(inputs.reshape(B * T), tabt)
    return out_flat.reshape(B, D, T)


# speedup vs baseline: 1.4720x; 1.4720x over previous
import functools

import jax
import jax.numpy as jnp
from jax import lax
from jax.experimental import pallas as pl
from jax.experimental.pallas import tpu as pltpu
from jax.experimental.pallas import tpu_sc as plsc

B = 1024
T = 200
D = 128
V = 1000

NC = 2
NS = 16
NW = NC * NS
BPW = B // NW

DC = 64
NCHUNK = D // DC
LANES = 16
NG = (T + LANES - 1) // LANES
NFULL = NG - 1
TAIL = T - NFULL * LANES
UNROLL = 8


def _make_sc_kernel():
    mesh = plsc.VectorSubcoreMesh(core_axis_name="c", subcore_axis_name="s")

    @functools.partial(
        pl.kernel,
        mesh=mesh,
        out_type=jax.ShapeDtypeStruct((B, D, T), jnp.float32),
        compiler_params=pltpu.CompilerParams(needs_layout_passes=False),
        scratch_types=[
            pltpu.VMEM((DC * V,), jnp.float32),
            pltpu.VMEM((BPW * T + LANES,), jnp.int32),
            pltpu.VMEM((DC, T), jnp.float32),
            pltpu.VMEM((DC, T), jnp.float32),
            pltpu.SemaphoreType.DMA,
            pltpu.SemaphoreType.DMA,
        ],
    )
    def k(idx_hbm, tabt_hbm, out_hbm, tab_v, idx_v, tile0, tile1, sem0, sem1):
        wid = lax.axis_index("s") * NC + lax.axis_index("c")
        lane = lax.iota(jnp.int32, 16)
        vstep = jnp.full((16,), V, jnp.int32)
        tstep = jnp.full((16,), T, jnp.int32)
        tail_mask = lane < TAIL

        idx_v[pl.ds(BPW * T, LANES)] = jnp.zeros((16,), jnp.int32)
        pltpu.sync_copy(
            idx_hbm.at[pl.ds(wid * BPW * T, BPW * T)],
            idx_v.at[pl.ds(0, BPW * T)],
        )

        def drain(tile, sem):
            pltpu.make_async_copy(
                tile, out_hbm.at[0, pl.ds(0, DC), :], sem
            ).wait()

        def gather_tile(i, tile):
            def body_g(g, _):
                idxv = idx_v[pl.ds(i * T + g * LANES, LANES)]

                def body_d(dd, _):
                    vals = plsc.load_gather(tab_v.at[pl.ds(dd * V, V)], [idxv])
                    tile[dd, pl.ds(g * LANES, LANES)] = vals
                    return 0

                lax.fori_loop(0, DC, body_d, 0, unroll=UNROLL)
                return 0

            lax.fori_loop(0, NFULL, body_g, 0)

            idxv = idx_v[pl.ds(i * T + NFULL * LANES, LANES)]
            tvec = NFULL * LANES + lane

            def body_dt(dd, _):
                dv = jnp.full((16,), dd, jnp.int32)
                vals = plsc.load_gather(tab_v.at[pl.ds(dd * V, V)], [idxv],
                                        mask=tail_mask)
                plsc.store_scatter(tile, [dv, tvec], vals, mask=tail_mask)
                return 0

            lax.fori_loop(0, DC, body_dt, 0, unroll=UNROLL)

        for chunk in range(NCHUNK):
            pltpu.sync_copy(tabt_hbm.at[pl.ds(chunk * DC * V, DC * V)], tab_v)

            def body_v(vv, _, chunk=chunk):
                for p, tile, sem in ((0, tile0, sem0), (1, tile1, sem1)):
                    if chunk == 0:
                        @pl.when(vv > 0)
                        def _():
                            drain(tile, sem)
                    else:
                        drain(tile, sem)
                    i = 2 * vv + p
                    gather_tile(i, tile)
                    b = wid * BPW + i
                    pltpu.async_copy(
                        tile,
                        out_hbm.at[b, pl.ds(chunk * DC, DC), :],
                        sem,
                    )
                return 0

            lax.fori_loop(0, BPW // 2, body_v, 0)

        drain(tile0, sem0)
        drain(tile1, sem1)

    return k


_sc_kernel = _make_sc_kernel()


def _transpose_body(x_ref, o_ref):
    o_ref[...] = jnp.transpose(x_ref[...], (1, 0))


_tc_transpose = pl.pallas_call(
    _transpose_body,
    out_shape=jax.ShapeDtypeStruct((D, V), jnp.float32),
)


def kernel(inputs, embedding_table):
    tabt = _tc_transpose(embedding_table).reshape(D * V)
    return _sc_kernel(inputs.reshape(B * T), tabt)


# speedup vs baseline: 2.6081x; 1.7718x over previous
import functools

import jax
import jax.numpy as jnp
from jax import lax
from jax.experimental import pallas as pl
from jax.experimental.pallas import tpu as pltpu
from jax.experimental.pallas import tpu_sc as plsc

B = 1024
T = 200
D = 128
V = 1000

NC = 2
NS = 16
NW = NC * NS
BPW = B // NW

DC = 64
NCHUNK = D // DC
LANES = 16
NG = (T + LANES - 1) // LANES
NFULL = NG - 1
TAIL = T - NFULL * LANES
UNROLL = 8


def _make_sc_kernel():
    mesh = plsc.VectorSubcoreMesh(core_axis_name="c", subcore_axis_name="s")

    @functools.partial(
        pl.kernel,
        mesh=mesh,
        out_type=jax.ShapeDtypeStruct((B, D, T), jnp.float32),
        compiler_params=pltpu.CompilerParams(needs_layout_passes=False),
        scratch_types=[
            pltpu.VMEM((DC * V,), jnp.float32),
            pltpu.VMEM((BPW * T + LANES,), jnp.int32),
            pltpu.VMEM((DC, T), jnp.float32),
            pltpu.VMEM((DC, T), jnp.float32),
            pltpu.SemaphoreType.DMA,
            pltpu.SemaphoreType.DMA,
        ],
    )
    def k(idx_hbm, tabt_hbm, out_hbm, tab_v, idx_v, tile0, tile1, sem0, sem1):
        wid = lax.axis_index("s") * NC + lax.axis_index("c")
        lane = lax.iota(jnp.int32, 16)
        vstep = jnp.full((16,), V, jnp.int32)
        tstep = jnp.full((16,), T, jnp.int32)
        tail_mask = lane < TAIL

        idx_v[pl.ds(BPW * T, LANES)] = jnp.zeros((16,), jnp.int32)
        pltpu.sync_copy(
            idx_hbm.at[pl.ds(wid * BPW * T, BPW * T)],
            idx_v.at[pl.ds(0, BPW * T)],
        )

        def drain(tile, sem):
            pltpu.make_async_copy(
                tile, out_hbm.at[0, pl.ds(0, DC), :], sem
            ).wait()

        def gather_tile(i, tile):
            def body_g(g, _):
                idxv = idx_v[pl.ds(i * T + g * LANES, LANES)]

                def body_d(blk, _):
                    d0 = blk * UNROLL
                    vals = [
                        plsc.load_gather(tab_v.at[pl.ds((d0 + u) * V, V)], [idxv])
                        for u in range(UNROLL)
                    ]
                    for u in range(UNROLL):
                        tile[d0 + u, pl.ds(g * LANES, LANES)] = vals[u]
                    return 0

                lax.fori_loop(0, DC // UNROLL, body_d, 0)
                return 0

            lax.fori_loop(0, NFULL, body_g, 0)

            idxv = idx_v[pl.ds(i * T + NFULL * LANES, LANES)]
            tvec = NFULL * LANES + lane

            def body_dt(blk, _):
                d0 = blk * UNROLL
                vals = [
                    plsc.load_gather(tab_v.at[pl.ds((d0 + u) * V, V)], [idxv],
                                     mask=tail_mask)
                    for u in range(UNROLL)
                ]
                for u in range(UNROLL):
                    dv = jnp.full((16,), d0 + u, jnp.int32)
                    plsc.store_scatter(tile, [dv, tvec], vals[u], mask=tail_mask)
                return 0

            lax.fori_loop(0, DC // UNROLL, body_dt, 0)

        for chunk in range(NCHUNK):
            pltpu.sync_copy(tabt_hbm.at[pl.ds(chunk * DC * V, DC * V)], tab_v)

            def body_v(vv, _, chunk=chunk):
                for p, tile, sem in ((0, tile0, sem0), (1, tile1, sem1)):
                    if chunk == 0:
                        @pl.when(vv > 0)
                        def _():
                            drain(tile, sem)
                    else:
                        drain(tile, sem)
                    i = 2 * vv + p
                    gather_tile(i, tile)
                    b = wid * BPW + i
                    pltpu.async_copy(
                        tile,
                        out_hbm.at[b, pl.ds(chunk * DC, DC), :],
                        sem,
                    )
                return 0

            lax.fori_loop(0, BPW // 2, body_v, 0)

        drain(tile0, sem0)
        drain(tile1, sem1)

    return k


_sc_kernel = _make_sc_kernel()


def _transpose_body(x_ref, o_ref):
    o_ref[...] = jnp.transpose(x_ref[...], (1, 0))


_tc_transpose = pl.pallas_call(
    _transpose_body,
    out_shape=jax.ShapeDtypeStruct((D, V), jnp.float32),
)


def kernel(inputs, embedding_table):
    tabt = _tc_transpose(embedding_table).reshape(D * V)
    return _sc_kernel(inputs.reshape(B * T), tabt)


# speedup vs baseline: 2.6087x; 1.0002x over previous
import functools

import jax
import jax.numpy as jnp
from jax import lax
from jax.experimental import pallas as pl
from jax.experimental.pallas import tpu as pltpu
from jax.experimental.pallas import tpu_sc as plsc

B = 1024
T = 200
D = 128
V = 1000

NC = 2
NS = 16
NW = NC * NS
BPW = B // NW

DC = 64
NCHUNK = D // DC
LANES = 16
NG = (T + LANES - 1) // LANES
NFULL = NG - 1
TAIL = T - NFULL * LANES
UNROLL = 8


def _make_sc_kernel():
    mesh = plsc.VectorSubcoreMesh(core_axis_name="c", subcore_axis_name="s")

    @functools.partial(
        pl.kernel,
        mesh=mesh,
        out_type=jax.ShapeDtypeStruct((B, D, T), jnp.float32),
        compiler_params=pltpu.CompilerParams(
            needs_layout_passes=False, use_tc_tiling_on_sc=True
        ),
        scratch_types=[
            pltpu.VMEM((DC * V,), jnp.float32),
            pltpu.VMEM((BPW * T + LANES,), jnp.int32),
            pltpu.VMEM((DC, T), jnp.float32),
            pltpu.VMEM((DC, T), jnp.float32),
            pltpu.SemaphoreType.DMA,
            pltpu.SemaphoreType.DMA,
        ],
    )
    def k(idx_hbm, tabt_hbm, out_hbm, tab_v, idx_v, tile0, tile1, sem0, sem1):
        wid = lax.axis_index("s") * NC + lax.axis_index("c")
        lane = lax.iota(jnp.int32, 16)
        vstep = jnp.full((16,), V, jnp.int32)
        tstep = jnp.full((16,), T, jnp.int32)
        tail_mask = lane < TAIL

        idx_v[pl.ds(BPW * T, LANES)] = jnp.zeros((16,), jnp.int32)
        pltpu.sync_copy(
            idx_hbm.at[pl.ds(wid * BPW * T, BPW * T)],
            idx_v.at[pl.ds(0, BPW * T)],
        )

        def drain(tile, sem):
            pltpu.make_async_copy(
                tile, out_hbm.at[0, pl.ds(0, DC), :], sem
            ).wait()

        def gather_tile(i, tile):
            def body_g(g, _):
                idxv = idx_v[pl.ds(i * T + g * LANES, LANES)]

                def body_d(blk, _):
                    d0 = blk * UNROLL
                    vals = [
                        plsc.load_gather(tab_v.at[pl.ds((d0 + u) * V, V)], [idxv])
                        for u in range(UNROLL)
                    ]
                    for u in range(UNROLL):
                        tile[d0 + u, pl.ds(g * LANES, LANES)] = vals[u]
                    return 0

                lax.fori_loop(0, DC // UNROLL, body_d, 0)
                return 0

            lax.fori_loop(0, NFULL, body_g, 0)

            idxv = idx_v[pl.ds(i * T + NFULL * LANES, LANES)]
            tvec = NFULL * LANES + lane

            def body_dt(blk, _):
                d0 = blk * UNROLL
                vals = [
                    plsc.load_gather(tab_v.at[pl.ds((d0 + u) * V, V)], [idxv],
                                     mask=tail_mask)
                    for u in range(UNROLL)
                ]
                for u in range(UNROLL):
                    dv = jnp.full((16,), d0 + u, jnp.int32)
                    plsc.store_scatter(tile, [dv, tvec], vals[u], mask=tail_mask)
                return 0

            lax.fori_loop(0, DC // UNROLL, body_dt, 0)

        for chunk in range(NCHUNK):
            pltpu.sync_copy(tabt_hbm.at[pl.ds(chunk * DC * V, DC * V)], tab_v)

            def body_v(vv, _, chunk=chunk):
                for p, tile, sem in ((0, tile0, sem0), (1, tile1, sem1)):
                    if chunk == 0:
                        @pl.when(vv > 0)
                        def _():
                            drain(tile, sem)
                    else:
                        drain(tile, sem)
                    i = 2 * vv + p
                    gather_tile(i, tile)
                    b = wid * BPW + i
                    pltpu.async_copy(
                        tile,
                        out_hbm.at[b, pl.ds(chunk * DC, DC), :],
                        sem,
                    )
                return 0

            lax.fori_loop(0, BPW // 2, body_v, 0)

        drain(tile0, sem0)
        drain(tile1, sem1)

    return k


_sc_kernel = _make_sc_kernel()


def _transpose_body(x_ref, o_ref):
    o_ref[...] = jnp.transpose(x_ref[...], (1, 0))


_tc_transpose = pl.pallas_call(
    _transpose_body,
    out_shape=jax.ShapeDtypeStruct((D, V), jnp.float32),
)


def kernel(inputs, embedding_table):
    tabt = _tc_transpose(embedding_table).reshape(D * V)
    return _sc_kernel(inputs.reshape(B * T), tabt)
